# baseline (speedup 1.0000x reference)
import functools

import jax
import jax.numpy as jnp
from jax import lax
from jax.experimental import pallas as pl
from jax.experimental.pallas import tpu as pltpu
from jax.experimental.pallas import tpu_sc as plsc

N = 10000
E = 320000
F = 128
NG = 64
NCLS = 16

NC = 2
NS = 16
NW = NC * NS

QE = 2048
QR = QE // 128
NQ0 = 5
NQ1 = 5
E_PAD = NS * QE * (NQ0 + NQ1)
EPW = E_PAD // NW
NQ_DEG = EPW // QE

CD = 2048
CD_R = CD // 128
CHUNKS_D = EPW // CD

_mesh = plsc.VectorSubcoreMesh(core_axis_name="c", subcore_axis_name="s")


@functools.partial(
    pl.kernel,
    mesh=_mesh,
    out_type=jax.ShapeDtypeStruct((NC, N), jnp.float32),
    scratch_types=[
        pltpu.VMEM((CD_R, 128), jnp.int32),
        pltpu.VMEM((CD,), jnp.float32),
        pltpu.VMEM_SHARED((N,), jnp.float32),
        pltpu.SemaphoreType.DMA,
    ],
)
def _deg_kernel(col2d_hbm, ew_hbm, zeros1_hbm, out_hbm, col_v, ew_v, acc, sem):
    cid = lax.axis_index("c")
    sid = lax.axis_index("s")
    wid = sid * NC + cid

    @pl.when(sid == 0)
    def _():
        pltpu.sync_copy(zeros1_hbm, acc)

    plsc.subcore_barrier()

    def chunk_body(t, carry):
        base_r = wid * (EPW // 128) + t * CD_R
        base_e = wid * EPW + t * CD
        pltpu.sync_copy(col2d_hbm.at[pl.ds(base_r, CD_R)], col_v)
        pltpu.sync_copy(ew_hbm.at[pl.ds(base_e, CD)], ew_v)
        for j in range(CD_R):
            pltpu.sync_copy(ew_v.at[pl.ds(j * 128, 128)],
                            acc.at[col_v.at[j]], add=True)
        return carry

    lax.fori_loop(0, CHUNKS_D, chunk_body, 0)
    plsc.subcore_barrier()

    @pl.when(sid == 0)
    def _():
        pltpu.sync_copy(acc, out_hbm.at[cid])


@functools.partial(
    pl.kernel,
    mesh=_mesh,
    out_type=jax.ShapeDtypeStruct((NC, N, F), jnp.float32),
    scratch_types=[
        pltpu.VMEM((QR, 128), jnp.int32),
        pltpu.VMEM((QR, 128), jnp.int32),
        pltpu.VMEM((QE,), jnp.float32),
        pltpu.VMEM((2, 128, F), jnp.float32),
        pltpu.VMEM_SHARED((N, F), jnp.float32),
        pltpu.SemaphoreType.DMA,
    ],
)
def _edge_kernel(y_hbm, row2d_hbm, col2d_hbm, ew_hbm, out_hbm,
                 row_v, col_v, ew_v, rows, acc, sem_g):
    cid = lax.axis_index("c")
    sid = lax.axis_index("s")
    nq = jnp.where(cid == 0, NQ0, NQ1)
    base_e_w = cid * (NS * NQ0 * QE) + sid * (nq * QE)
    base_r_w = cid * (NS * NQ0 * QR) + sid * (nq * QR)

    zrow = jnp.zeros((16,), jnp.float32)

    def zrow_body(r, c):
        for qf in range(F // 16):
            rows[0, r, pl.ds(qf * 16, 16)] = zrow
        return c

    lax.fori_loop(0, 16, zrow_body, 0)
    nz = jnp.where(sid == NS - 1, 25, 40)

    def zcp_body(k, c):
        pltpu.sync_copy(rows.at[0, pl.ds(0, 16)],
                        acc.at[pl.ds(sid * 640 + k * 16, 16)])
        return c

    lax.fori_loop(0, nz, zcp_body, 0)
    plsc.subcore_barrier()

    def quarter_body(q, carry):
        base_r = base_r_w + q * QR
        base_e = base_e_w + q * QE
        pltpu.sync_copy(row2d_hbm.at[pl.ds(base_r, QR)], row_v)
        pltpu.sync_copy(col2d_hbm.at[pl.ds(base_r, QR)], col_v)
        pltpu.sync_copy(ew_hbm.at[pl.ds(base_e, QE)], ew_v)

        pltpu.async_copy(y_hbm.at[row_v.at[0]], rows.at[0], sem_g)

        def pair_body(hs, c2):
            for j2 in range(2):
                t = hs * 2 + j2
                par = j2
                pltpu.make_async_copy(y_hbm.at[row_v.at[0]],
                                      rows.at[par], sem_g).wait()

                @pl.when(t < QR - 1)
                def _():
                    pltpu.async_copy(y_hbm.at[row_v.at[t + 1]],
                                     rows.at[1 - par], sem_g)

                def group_body(g, c3):
                    wv = ew_v[pl.ds(t * 128 + g * 16, 16)]
                    for l in range(16):
                        w = wv[l]
                        for qf in range(F // 16):
                            sl = pl.ds(qf * 16, 16)
                            rows[par, g * 16 + l, sl] = \
                                rows[par, g * 16 + l, sl] * w
                    return c3

                lax.fori_loop(0, 8, group_body, 0)

                pltpu.sync_copy(rows.at[par], acc.at[col_v.at[t]], add=True)
            return c2

        lax.fori_loop(0, QR // 2, pair_body, 0)
        return carry

    lax.fori_loop(0, nq, quarter_body, 0)
    plsc.subcore_barrier()

    @pl.when(sid < NS - 1)
    def _():
        pltpu.sync_copy(acc.at[pl.ds(sid * 640, 640)],
                        out_hbm.at[cid, pl.ds(sid * 640, 640)])

    @pl.when(sid == NS - 1)
    def _():
        pltpu.sync_copy(acc.at[pl.ds(9600, 400)],
                        out_hbm.at[cid, pl.ds(9600, 400)])


_BLK = 1000
_G = N // _BLK


def _dis_of(dega, degb):
    deg = dega + degb + 1.0
    return jnp.where(deg > 0, lax.rsqrt(jnp.maximum(deg, 1e-12)), 0.0)


def _y_body(x_ref, w_ref, dega_ref, degb_ref, y_ref):
    dis = _dis_of(dega_ref[...], degb_ref[...])
    y_ref[...] = (x_ref[...] @ w_ref[...]) * dis


def _mid_body(s1a_ref, s1b_ref, y1_ref, dega_ref, degb_ref, w2_ref, b1_ref,
              y2_ref):
    dis = _dis_of(dega_ref[...], degb_ref[...])
    h1 = dis * (s1a_ref[...] + s1b_ref[...] + y1_ref[...]) + b1_ref[...]
    h1 = jnp.maximum(h1, 0.0)
    y2_ref[...] = (h1 @ w2_ref[...]) * dis


def _pool_body(s2a_ref, s2b_ref, y2_ref, dega_ref, degb_ref, b2_ref,
               batch_ref, wlin_ref, blin_ref, out_ref, pooled_acc, counts_acc):
    i = pl.program_id(0)

    @pl.when(i == 0)
    def _():
        pooled_acc[...] = jnp.zeros_like(pooled_acc)
        counts_acc[...] = jnp.zeros_like(counts_acc)

    dis = _dis_of(dega_ref[...], degb_ref[...])
    h2 = dis * (s2a_ref[...] + s2b_ref[...] + y2_ref[...]) + b2_ref[...]
    gids = lax.broadcasted_iota(jnp.int32, (_BLK, NG), 1)
    oh = (batch_ref[...] == gids).astype(jnp.float32)
    dn = (((0,), (0,)), ((), ()))
    pooled_acc[...] += lax.dot_general(
        oh, h2, dn, preferred_element_type=jnp.float32)
    counts_acc[...] += lax.dot_general(
        oh, jnp.ones((_BLK, F), jnp.float32), dn,
        preferred_element_type=jnp.float32)

    @pl.when(i == _G - 1)
    def _():
        pooled = pooled_acc[...] / jnp.maximum(counts_acc[...], 1.0)
        out_ref[...] = pooled @ wlin_ref[...] + blin_ref[...]


def _row_spec(width):
    return pl.BlockSpec((_BLK, width), lambda i: (i, 0))


def _full_spec(shape):
    nd = len(shape)
    return pl.BlockSpec(shape, lambda i: (0,) * nd)


def kernel(x, edge_index, edge_weight, batch, W1, b1, W2, b2, Wlin, blin):
    row = jnp.pad(edge_index[0], (0, E_PAD - E))
    col = jnp.pad(edge_index[1], (0, E_PAD - E))
    ew = jnp.pad(edge_weight, (0, E_PAD - E))
    row2d = row.reshape(E_PAD // 128, 128)
    col2d = col.reshape(E_PAD // 128, 128)
    zeros1 = jnp.zeros((N,), jnp.float32)

    degp = _deg_kernel(col2d, ew, zeros1)
    dega = degp[0][:, None]
    degb = degp[1][:, None]

    y1 = pl.pallas_call(
        _y_body,
        grid=(_G,),
        in_specs=[_row_spec(F), _full_spec((F, F)), _row_spec(1), _row_spec(1)],
        out_specs=_row_spec(F),
        out_shape=jax.ShapeDtypeStruct((N, F), jnp.float32),
    )(x, W1, dega, degb)

    s1 = _edge_kernel(y1, row2d, col2d, ew)

    y2 = pl.pallas_call(
        _mid_body,
        grid=(_G,),
        in_specs=[_row_spec(F), _row_spec(F), _row_spec(F), _row_spec(1),
                  _row_spec(1), _full_spec((F, F)), _full_spec((1, F))],
        out_specs=_row_spec(F),
        out_shape=jax.ShapeDtypeStruct((N, F), jnp.float32),
    )(s1[0], s1[1], y1, dega, degb, W2, b1[None, :])

    s2 = _edge_kernel(y2, row2d, col2d, ew)

    out = pl.pallas_call(
        _pool_body,
        grid=(_G,),
        in_specs=[_row_spec(F), _row_spec(F), _row_spec(F), _row_spec(1),
                  _row_spec(1), _full_spec((1, F)), _row_spec(1),
                  _full_spec((F, NCLS)), _full_spec((1, NCLS))],
        out_specs=_full_spec((NG, NCLS)),
        out_shape=jax.ShapeDtypeStruct((NG, NCLS), jnp.float32),
        scratch_shapes=[pltpu.VMEM((NG, F), jnp.float32),
                        pltpu.VMEM((NG, F), jnp.float32)],
    )(s2[0], s2[1], y2, dega, degb, b2[None, :], batch[:, None],
      Wlin, blin[None, :])

    return out


# speedup vs baseline: 2.9447x; 2.9447x over previous
import functools

import jax
import jax.numpy as jnp
from jax import lax
from jax.experimental import pallas as pl
from jax.experimental.pallas import tpu as pltpu
from jax.experimental.pallas import tpu_sc as plsc

N = 10000
E = 320000
F = 128
NG = 64
NCLS = 16

NC = 2
NS = 16
NW = NC * NS

QE = 2048
QR = QE // 128
NQ0 = 5
NQ1 = 5
E_PAD = NS * QE * (NQ0 + NQ1)
EPW = E_PAD // NW
NQ_DEG = EPW // QE

CD = 2048
CD_R = CD // 128
CHUNKS_D = EPW // CD

_mesh = plsc.VectorSubcoreMesh(core_axis_name="c", subcore_axis_name="s")


@functools.partial(
    pl.kernel,
    mesh=_mesh,
    out_type=jax.ShapeDtypeStruct((NC, N), jnp.float32),
    scratch_types=[
        pltpu.VMEM((CD_R, 128), jnp.int32),
        pltpu.VMEM((CD,), jnp.float32),
        pltpu.VMEM_SHARED((N,), jnp.float32),
        pltpu.SemaphoreType.DMA,
    ],
)
def _deg_kernel(col2d_hbm, ew_hbm, zeros1_hbm, out_hbm, col_v, ew_v, acc, sem):
    cid = lax.axis_index("c")
    sid = lax.axis_index("s")
    wid = sid * NC + cid

    @pl.when(sid == 0)
    def _():
        pltpu.sync_copy(zeros1_hbm, acc)

    plsc.subcore_barrier()

    def chunk_body(t, carry):
        base_r = wid * (EPW // 128) + t * CD_R
        base_e = wid * EPW + t * CD
        pltpu.sync_copy(col2d_hbm.at[pl.ds(base_r, CD_R)], col_v)
        pltpu.sync_copy(ew_hbm.at[pl.ds(base_e, CD)], ew_v)
        for j in range(CD_R):
            pltpu.sync_copy(ew_v.at[pl.ds(j * 128, 128)],
                            acc.at[col_v.at[j]], add=True)
        return carry

    lax.fori_loop(0, CHUNKS_D, chunk_body, 0)
    plsc.subcore_barrier()

    @pl.when(sid == 0)
    def _():
        pltpu.sync_copy(acc, out_hbm.at[cid])


@functools.partial(
    pl.kernel,
    mesh=_mesh,
    out_type=jax.ShapeDtypeStruct((NC, N, F), jnp.float32),
    scratch_types=[
        pltpu.VMEM((QR, 128), jnp.int32),
        pltpu.VMEM((QR, 128), jnp.int32),
        pltpu.VMEM((QE,), jnp.float32),
        pltpu.VMEM((2, 128, F), jnp.float32),
        pltpu.VMEM_SHARED((N, F), jnp.float32),
        pltpu.SemaphoreType.DMA,
    ],
)
def _edge_kernel(y_hbm, row2d_hbm, col2d_hbm, ew_hbm, out_hbm,
                 row_v, col_v, ew_v, rows, acc, sem_g):
    cid = lax.axis_index("c")
    sid = lax.axis_index("s")
    nq = jnp.where(cid == 0, NQ0, NQ1)
    base_e_w = cid * (NS * NQ0 * QE) + sid * (nq * QE)
    base_r_w = cid * (NS * NQ0 * QR) + sid * (nq * QR)

    zrow = jnp.zeros((16,), jnp.float32)

    def zrow_body(r, c):
        for qf in range(F // 16):
            rows[0, r, pl.ds(qf * 16, 16)] = zrow
        return c

    lax.fori_loop(0, 16, zrow_body, 0)
    nz = jnp.where(sid == NS - 1, 25, 40)

    def zcp_body(k, c):
        pltpu.sync_copy(rows.at[0, pl.ds(0, 16)],
                        acc.at[pl.ds(sid * 640 + k * 16, 16)])
        return c

    lax.fori_loop(0, nz, zcp_body, 0)
    plsc.subcore_barrier()

    def quarter_body(q, carry):
        base_r = base_r_w + q * QR
        base_e = base_e_w + q * QE
        pltpu.sync_copy(row2d_hbm.at[pl.ds(base_r, QR)], row_v)
        pltpu.sync_copy(col2d_hbm.at[pl.ds(base_r, QR)], col_v)
        pltpu.sync_copy(ew_hbm.at[pl.ds(base_e, QE)], ew_v)

        pltpu.async_copy(y_hbm.at[row_v.at[0]], rows.at[0], sem_g)

        def pair_body(hs, c2):
            for j2 in range(2):
                t = hs * 2 + j2
                par = j2
                pltpu.make_async_copy(y_hbm.at[row_v.at[0]],
                                      rows.at[par], sem_g).wait()

                @pl.when(t < QR - 1)
                def _():
                    pltpu.async_copy(y_hbm.at[row_v.at[t + 1]],
                                     rows.at[1 - par], sem_g)

                def group_body(g, c3):
                    wv = ew_v[pl.ds(t * 128 + g * 16, 16)]
                    for l in range(16):
                        w = wv[l]
                        for qf in range(F // 16):
                            sl = pl.ds(qf * 16, 16)
                            rows[par, g * 16 + l, sl] = \
                                rows[par, g * 16 + l, sl] * w
                    return c3

                lax.fori_loop(0, 8, group_body, 0)

                pltpu.sync_copy(rows.at[par], acc.at[col_v.at[t]], add=True)
            return c2

        lax.fori_loop(0, QR // 2, pair_body, 0)
        return carry

    lax.fori_loop(0, nq, quarter_body, 0)
    plsc.subcore_barrier()

    @pl.when(sid < NS - 1)
    def _():
        pltpu.sync_copy(acc.at[pl.ds(sid * 640, 640)],
                        out_hbm.at[cid, pl.ds(sid * 640, 640)])

    @pl.when(sid == NS - 1)
    def _():
        pltpu.sync_copy(acc.at[pl.ds(9600, 400)],
                        out_hbm.at[cid, pl.ds(9600, 400)])


_BLK = 1000
_G = N // _BLK


def _dis_of(dega, degb):
    deg = dega + degb + 1.0
    return jnp.where(deg > 0, lax.rsqrt(jnp.maximum(deg, 1e-12)), 0.0)


def _y_body(x_ref, w_ref, dega_ref, degb_ref, y_ref):
    dis = _dis_of(dega_ref[...], degb_ref[...])
    y_ref[...] = (x_ref[...] @ w_ref[...]) * dis


def _mid_body(s1a_ref, s1b_ref, y1_ref, dega_ref, degb_ref, w2_ref, b1_ref,
              y2_ref):
    dis = _dis_of(dega_ref[...], degb_ref[...])
    h1 = dis * (s1a_ref[...] + s1b_ref[...] + y1_ref[...]) + b1_ref[...]
    h1 = jnp.maximum(h1, 0.0)
    y2_ref[...] = (h1 @ w2_ref[...]) * dis


def _pool_body(s2a_ref, s2b_ref, y2_ref, dega_ref, degb_ref, b2_ref,
               batch_ref, wlin_ref, blin_ref, out_ref, pooled_acc, counts_acc):
    i = pl.program_id(0)

    @pl.when(i == 0)
    def _():
        pooled_acc[...] = jnp.zeros_like(pooled_acc)
        counts_acc[...] = jnp.zeros_like(counts_acc)

    dis = _dis_of(dega_ref[...], degb_ref[...])
    h2 = dis * (s2a_ref[...] + s2b_ref[...] + y2_ref[...]) + b2_ref[...]
    gids = lax.broadcasted_iota(jnp.int32, (_BLK, NG), 1)
    oh = (batch_ref[...] == gids).astype(jnp.float32)
    dn = (((0,), (0,)), ((), ()))
    pooled_acc[...] += lax.dot_general(
        oh, h2, dn, preferred_element_type=jnp.float32)
    counts_acc[...] += lax.dot_general(
        oh, jnp.ones((_BLK, F), jnp.float32), dn,
        preferred_element_type=jnp.float32)

    @pl.when(i == _G - 1)
    def _():
        pooled = pooled_acc[...] / jnp.maximum(counts_acc[...], 1.0)
        out_ref[...] = pooled @ wlin_ref[...] + blin_ref[...]


def _row_spec(width):
    return pl.BlockSpec((_BLK, width), lambda i: (i, 0))


def _full_spec(shape):
    nd = len(shape)
    return pl.BlockSpec(shape, lambda i: (0,) * nd)


def kernel(x, edge_index, edge_weight, batch, W1, b1, W2, b2, Wlin, blin):
    pad_idx = jnp.arange(E_PAD - E, dtype=jnp.int32) % N
    row = jnp.concatenate([edge_index[0], pad_idx])
    col = jnp.concatenate([edge_index[1], pad_idx])
    ew = jnp.pad(edge_weight, (0, E_PAD - E))
    row2d = row.reshape(E_PAD // 128, 128)
    col2d = col.reshape(E_PAD // 128, 128)
    zeros1 = jnp.zeros((N,), jnp.float32)

    degp = _deg_kernel(col2d, ew, zeros1)
    dega = degp[0][:, None]
    degb = degp[1][:, None]

    y1 = pl.pallas_call(
        _y_body,
        grid=(_G,),
        in_specs=[_row_spec(F), _full_spec((F, F)), _row_spec(1), _row_spec(1)],
        out_specs=_row_spec(F),
        out_shape=jax.ShapeDtypeStruct((N, F), jnp.float32),
    )(x, W1, dega, degb)

    s1 = _edge_kernel(y1, row2d, col2d, ew)

    y2 = pl.pallas_call(
        _mid_body,
        grid=(_G,),
        in_specs=[_row_spec(F), _row_spec(F), _row_spec(F), _row_spec(1),
                  _row_spec(1), _full_spec((F, F)), _full_spec((1, F))],
        out_specs=_row_spec(F),
        out_shape=jax.ShapeDtypeStruct((N, F), jnp.float32),
    )(s1[0], s1[1], y1, dega, degb, W2, b1[None, :])

    s2 = _edge_kernel(y2, row2d, col2d, ew)

    out = pl.pallas_call(
        _pool_body,
        grid=(_G,),
        in_specs=[_row_spec(F), _row_spec(F), _row_spec(F), _row_spec(1),
                  _row_spec(1), _full_spec((1, F)), _row_spec(1),
                  _full_spec((F, NCLS)), _full_spec((1, NCLS))],
        out_specs=_full_spec((NG, NCLS)),
        out_shape=jax.ShapeDtypeStruct((NG, NCLS), jnp.float32),
        scratch_shapes=[pltpu.VMEM((NG, F), jnp.float32),
                        pltpu.VMEM((NG, F), jnp.float32)],
    )(s2[0], s2[1], y2, dega, degb, b2[None, :], batch[:, None],
      Wlin, blin[None, :])

    return out


# speedup vs baseline: 2.9571x; 1.0042x over previous
import functools

import jax
import jax.numpy as jnp
from jax import lax
from jax.experimental import pallas as pl
from jax.experimental.pallas import tpu as pltpu
from jax.experimental.pallas import tpu_sc as plsc

N = 10000
E = 320000
F = 128
NG = 64
NCLS = 16

NC = 2
NS = 16
NW = NC * NS

QE = 2048
QR = QE // 128
NQ0 = 5
NQ1 = 5
E_PAD = NS * QE * (NQ0 + NQ1)
EPW = E_PAD // NW
NQ_DEG = EPW // QE

CD = 2048
CD_R = CD // 128
CHUNKS_D = EPW // CD

_mesh = plsc.VectorSubcoreMesh(core_axis_name="c", subcore_axis_name="s")


@functools.partial(
    pl.kernel,
    mesh=_mesh,
    out_type=jax.ShapeDtypeStruct((NC, N), jnp.float32),
    scratch_types=[
        pltpu.VMEM((CD_R, 128), jnp.int32),
        pltpu.VMEM((CD,), jnp.float32),
        pltpu.VMEM_SHARED((N,), jnp.float32),
        pltpu.SemaphoreType.DMA,
    ],
)
def _deg_kernel(col2d_hbm, ew_hbm, zeros1_hbm, out_hbm, col_v, ew_v, acc, sem):
    cid = lax.axis_index("c")
    sid = lax.axis_index("s")
    wid = sid * NC + cid

    @pl.when(sid == 0)
    def _():
        pltpu.sync_copy(zeros1_hbm, acc)

    plsc.subcore_barrier()

    def chunk_body(t, carry):
        base_r = wid * (EPW // 128) + t * CD_R
        base_e = wid * EPW + t * CD
        pltpu.sync_copy(col2d_hbm.at[pl.ds(base_r, CD_R)], col_v)
        pltpu.sync_copy(ew_hbm.at[pl.ds(base_e, CD)], ew_v)
        for j in range(CD_R):
            pltpu.sync_copy(ew_v.at[pl.ds(j * 128, 128)],
                            acc.at[col_v.at[j]], add=True)
        return carry

    lax.fori_loop(0, CHUNKS_D, chunk_body, 0)
    plsc.subcore_barrier()

    @pl.when(sid == 0)
    def _():
        pltpu.sync_copy(acc, out_hbm.at[cid])


@functools.partial(
    pl.kernel,
    mesh=_mesh,
    out_type=jax.ShapeDtypeStruct((NC, N, F), jnp.float32),
    scratch_types=[
        pltpu.VMEM((QR, 128), jnp.int32),
        pltpu.VMEM((QR, 128), jnp.int32),
        pltpu.VMEM((QE,), jnp.float32),
        pltpu.VMEM((2, 128, F), jnp.float32),
        pltpu.VMEM_SHARED((N, F), jnp.float32),
        pltpu.SemaphoreType.DMA,
    ],
)
def _edge_kernel(y_hbm, row2d_hbm, col2d_hbm, ew_hbm, out_hbm,
                 row_v, col_v, ew_v, rows, acc, sem_g):
    cid = lax.axis_index("c")
    sid = lax.axis_index("s")
    nq = jnp.where(cid == 0, NQ0, NQ1)
    base_e_w = cid * (NS * NQ0 * QE) + sid * (nq * QE)
    base_r_w = cid * (NS * NQ0 * QR) + sid * (nq * QR)

    zrow = jnp.zeros((16,), jnp.float32)

    def zrow_body(r, c):
        for qf in range(F // 16):
            rows[0, r, pl.ds(qf * 16, 16)] = zrow
        return c

    lax.fori_loop(0, 16, zrow_body, 0)
    nz = jnp.where(sid == NS - 1, 25, 40)

    def zcp_body(k, c):
        pltpu.sync_copy(rows.at[0, pl.ds(0, 16)],
                        acc.at[pl.ds(sid * 640 + k * 16, 16)])
        return c

    lax.fori_loop(0, nz, zcp_body, 0)
    plsc.subcore_barrier()

    def quarter_body(q, carry):
        base_r = base_r_w + q * QR
        base_e = base_e_w + q * QE
        pltpu.sync_copy(row2d_hbm.at[pl.ds(base_r, QR)], row_v)
        pltpu.sync_copy(col2d_hbm.at[pl.ds(base_r, QR)], col_v)
        pltpu.sync_copy(ew_hbm.at[pl.ds(base_e, QE)], ew_v)

        pltpu.async_copy(y_hbm.at[row_v.at[0]], rows.at[0], sem_g)

        def pair_body(hs, c2):
            for j2 in range(2):
                t = hs * 2 + j2
                par = j2
                pltpu.make_async_copy(y_hbm.at[row_v.at[0]],
                                      rows.at[par], sem_g).wait()

                @pl.when(t < QR - 1)
                def _():
                    pltpu.async_copy(y_hbm.at[row_v.at[t + 1]],
                                     rows.at[1 - par], sem_g)

                @plsc.parallel_loop(0, 8, unroll=2)
                def _(g):
                    wv = ew_v[pl.ds(t * 128 + g * 16, 16)]
                    for l in range(16):
                        w = wv[l]
                        for qf in range(F // 16):
                            sl = pl.ds(qf * 16, 16)
                            rows[par, g * 16 + l, sl] = \
                                rows[par, g * 16 + l, sl] * w

                pltpu.sync_copy(rows.at[par], acc.at[col_v.at[t]], add=True)
            return c2

        lax.fori_loop(0, QR // 2, pair_body, 0)
        return carry

    lax.fori_loop(0, nq, quarter_body, 0)
    plsc.subcore_barrier()

    @pl.when(sid < NS - 1)
    def _():
        pltpu.sync_copy(acc.at[pl.ds(sid * 640, 640)],
                        out_hbm.at[cid, pl.ds(sid * 640, 640)])

    @pl.when(sid == NS - 1)
    def _():
        pltpu.sync_copy(acc.at[pl.ds(9600, 400)],
                        out_hbm.at[cid, pl.ds(9600, 400)])


_BLK = 1000
_G = N // _BLK


def _dis_of(dega, degb):
    deg = dega + degb + 1.0
    return jnp.where(deg > 0, lax.rsqrt(jnp.maximum(deg, 1e-12)), 0.0)


def _y_body(x_ref, w_ref, dega_ref, degb_ref, y_ref):
    dis = _dis_of(dega_ref[...], degb_ref[...])
    y_ref[...] = (x_ref[...] @ w_ref[...]) * dis


def _mid_body(s1a_ref, s1b_ref, y1_ref, dega_ref, degb_ref, w2_ref, b1_ref,
              y2_ref):
    dis = _dis_of(dega_ref[...], degb_ref[...])
    h1 = dis * (s1a_ref[...] + s1b_ref[...] + y1_ref[...]) + b1_ref[...]
    h1 = jnp.maximum(h1, 0.0)
    y2_ref[...] = (h1 @ w2_ref[...]) * dis


def _pool_body(s2a_ref, s2b_ref, y2_ref, dega_ref, degb_ref, b2_ref,
               batch_ref, wlin_ref, blin_ref, out_ref, pooled_acc, counts_acc):
    i = pl.program_id(0)

    @pl.when(i == 0)
    def _():
        pooled_acc[...] = jnp.zeros_like(pooled_acc)
        counts_acc[...] = jnp.zeros_like(counts_acc)

    dis = _dis_of(dega_ref[...], degb_ref[...])
    h2 = dis * (s2a_ref[...] + s2b_ref[...] + y2_ref[...]) + b2_ref[...]
    gids = lax.broadcasted_iota(jnp.int32, (_BLK, NG), 1)
    oh = (batch_ref[...] == gids).astype(jnp.float32)
    dn = (((0,), (0,)), ((), ()))
    pooled_acc[...] += lax.dot_general(
        oh, h2, dn, preferred_element_type=jnp.float32)
    counts_acc[...] += lax.dot_general(
        oh, jnp.ones((_BLK, F), jnp.float32), dn,
        preferred_element_type=jnp.float32)

    @pl.when(i == _G - 1)
    def _():
        pooled = pooled_acc[...] / jnp.maximum(counts_acc[...], 1.0)
        out_ref[...] = pooled @ wlin_ref[...] + blin_ref[...]


def _row_spec(width):
    return pl.BlockSpec((_BLK, width), lambda i: (i, 0))


def _full_spec(shape):
    nd = len(shape)
    return pl.BlockSpec(shape, lambda i: (0,) * nd)


def kernel(x, edge_index, edge_weight, batch, W1, b1, W2, b2, Wlin, blin):
    pad_idx = jnp.arange(E_PAD - E, dtype=jnp.int32) % N
    row = jnp.concatenate([edge_index[0], pad_idx])
    col = jnp.concatenate([edge_index[1], pad_idx])
    ew = jnp.pad(edge_weight, (0, E_PAD - E))
    row2d = row.reshape(E_PAD // 128, 128)
    col2d = col.reshape(E_PAD // 128, 128)
    zeros1 = jnp.zeros((N,), jnp.float32)

    degp = _deg_kernel(col2d, ew, zeros1)
    dega = degp[0][:, None]
    degb = degp[1][:, None]

    y1 = pl.pallas_call(
        _y_body,
        grid=(_G,),
        in_specs=[_row_spec(F), _full_spec((F, F)), _row_spec(1), _row_spec(1)],
        out_specs=_row_spec(F),
        out_shape=jax.ShapeDtypeStruct((N, F), jnp.float32),
    )(x, W1, dega, degb)

    s1 = _edge_kernel(y1, row2d, col2d, ew)

    y2 = pl.pallas_call(
        _mid_body,
        grid=(_G,),
        in_specs=[_row_spec(F), _row_spec(F), _row_spec(F), _row_spec(1),
                  _row_spec(1), _full_spec((F, F)), _full_spec((1, F))],
        out_specs=_row_spec(F),
        out_shape=jax.ShapeDtypeStruct((N, F), jnp.float32),
    )(s1[0], s1[1], y1, dega, degb, W2, b1[None, :])

    s2 = _edge_kernel(y2, row2d, col2d, ew)

    out = pl.pallas_call(
        _pool_body,
        grid=(_G,),
        in_specs=[_row_spec(F), _row_spec(F), _row_spec(F), _row_spec(1),
                  _row_spec(1), _full_spec((1, F)), _row_spec(1),
                  _full_spec((F, NCLS)), _full_spec((1, NCLS))],
        out_specs=_full_spec((NG, NCLS)),
        out_shape=jax.ShapeDtypeStruct((NG, NCLS), jnp.float32),
        scratch_shapes=[pltpu.VMEM((NG, F), jnp.float32),
                        pltpu.VMEM((NG, F), jnp.float32)],
    )(s2[0], s2[1], y2, dega, degb, b2[None, :], batch[:, None],
      Wlin, blin[None, :])

    return out


# speedup vs baseline: 2.9722x; 1.0051x over previous
import functools

import jax
import jax.numpy as jnp
from jax import lax
from jax.experimental import pallas as pl
from jax.experimental.pallas import tpu as pltpu
from jax.experimental.pallas import tpu_sc as plsc

N = 10000
E = 320000
F = 128
NG = 64
NCLS = 16

NC = 2
NS = 16
NW = NC * NS

QE = 2048
QR = QE // 128
NQ0 = 5
NQ1 = 5
E_PAD = NS * QE * (NQ0 + NQ1)
EPW = E_PAD // NW
NQ_DEG = EPW // QE

CD = 2048
CD_R = CD // 128
CHUNKS_D = EPW // CD

_mesh = plsc.VectorSubcoreMesh(core_axis_name="c", subcore_axis_name="s")


@functools.partial(
    pl.kernel,
    mesh=_mesh,
    out_type=jax.ShapeDtypeStruct((NC, N), jnp.float32),
    scratch_types=[
        pltpu.VMEM((CD_R, 128), jnp.int32),
        pltpu.VMEM((CD,), jnp.float32),
        pltpu.VMEM_SHARED((N,), jnp.float32),
        pltpu.SemaphoreType.DMA,
    ],
)
def _deg_kernel(col2d_hbm, ew_hbm, zeros1_hbm, out_hbm, col_v, ew_v, acc, sem):
    cid = lax.axis_index("c")
    sid = lax.axis_index("s")
    wid = sid * NC + cid

    @pl.when(sid == 0)
    def _():
        pltpu.sync_copy(zeros1_hbm, acc)

    plsc.subcore_barrier()

    def chunk_body(t, carry):
        base_r = wid * (EPW // 128) + t * CD_R
        base_e = wid * EPW + t * CD
        pltpu.sync_copy(col2d_hbm.at[pl.ds(base_r, CD_R)], col_v)
        pltpu.sync_copy(ew_hbm.at[pl.ds(base_e, CD)], ew_v)
        for j in range(CD_R):
            pltpu.sync_copy(ew_v.at[pl.ds(j * 128, 128)],
                            acc.at[col_v.at[j]], add=True)
        return carry

    lax.fori_loop(0, CHUNKS_D, chunk_body, 0)
    plsc.subcore_barrier()

    @pl.when(sid == 0)
    def _():
        pltpu.sync_copy(acc, out_hbm.at[cid])


@functools.partial(
    pl.kernel,
    mesh=_mesh,
    out_type=jax.ShapeDtypeStruct((NC, N, F), jnp.float32),
    scratch_types=[
        pltpu.VMEM((QR, 128), jnp.int32),
        pltpu.VMEM((QR, 128), jnp.int32),
        pltpu.VMEM((QE,), jnp.float32),
        pltpu.VMEM((2, 128, F), jnp.float32),
        pltpu.VMEM_SHARED((N, F), jnp.float32),
        pltpu.SemaphoreType.DMA,
        pltpu.SemaphoreType.DMA,
    ],
)
def _edge_kernel(y_hbm, row2d_hbm, col2d_hbm, ew_hbm, out_hbm,
                 row_v, col_v, ew_v, rows, acc, sem_g, sem_s):
    cid = lax.axis_index("c")
    sid = lax.axis_index("s")
    nq = jnp.where(cid == 0, NQ0, NQ1)
    base_e_w = cid * (NS * NQ0 * QE) + sid * (nq * QE)
    base_r_w = cid * (NS * NQ0 * QR) + sid * (nq * QR)

    zrow = jnp.zeros((16,), jnp.float32)

    def zrow_body(r, c):
        for qf in range(F // 16):
            rows[0, r, pl.ds(qf * 16, 16)] = zrow
        return c

    lax.fori_loop(0, 16, zrow_body, 0)
    nz = jnp.where(sid == NS - 1, 25, 40)

    def zcp_body(k, c):
        pltpu.sync_copy(rows.at[0, pl.ds(0, 16)],
                        acc.at[pl.ds(sid * 640 + k * 16, 16)])
        return c

    lax.fori_loop(0, nz, zcp_body, 0)
    plsc.subcore_barrier()

    def quarter_body(q, carry):
        base_r = base_r_w + q * QR
        base_e = base_e_w + q * QE
        pltpu.sync_copy(row2d_hbm.at[pl.ds(base_r, QR)], row_v)
        pltpu.sync_copy(col2d_hbm.at[pl.ds(base_r, QR)], col_v)
        pltpu.sync_copy(ew_hbm.at[pl.ds(base_e, QE)], ew_v)

        pltpu.async_copy(y_hbm.at[row_v.at[0]], rows.at[0], sem_g)

        def pair_body(hs, c2):
            for j2 in range(2):
                t = hs * 2 + j2
                par = j2
                pltpu.make_async_copy(y_hbm.at[row_v.at[0]],
                                      rows.at[par], sem_g).wait()

                prev_exists = (hs > 0) | (q > 0) if j2 == 0 else None

                def _wait_prev():
                    pltpu.make_async_copy(rows.at[1 - par],
                                          acc.at[col_v.at[0]], sem_s).wait()

                if j2 == 0:
                    pl.when(prev_exists)(_wait_prev)
                else:
                    _wait_prev()

                @pl.when(t < QR - 1)
                def _():
                    pltpu.async_copy(y_hbm.at[row_v.at[t + 1]],
                                     rows.at[1 - par], sem_g)

                @plsc.parallel_loop(0, 8, unroll=2)
                def _(g):
                    wv = ew_v[pl.ds(t * 128 + g * 16, 16)]
                    for l in range(16):
                        w = wv[l]
                        for qf in range(F // 16):
                            sl = pl.ds(qf * 16, 16)
                            rows[par, g * 16 + l, sl] = \
                                rows[par, g * 16 + l, sl] * w

                pltpu.async_copy(rows.at[par], acc.at[col_v.at[t]], sem_s,
                                 add=True)
            return c2

        lax.fori_loop(0, QR // 2, pair_body, 0)
        return carry

    lax.fori_loop(0, nq, quarter_body, 0)
    pltpu.make_async_copy(rows.at[1], acc.at[col_v.at[0]], sem_s).wait()
    plsc.subcore_barrier()

    @pl.when(sid < NS - 1)
    def _():
        pltpu.sync_copy(acc.at[pl.ds(sid * 640, 640)],
                        out_hbm.at[cid, pl.ds(sid * 640, 640)])

    @pl.when(sid == NS - 1)
    def _():
        pltpu.sync_copy(acc.at[pl.ds(9600, 400)],
                        out_hbm.at[cid, pl.ds(9600, 400)])


_BLK = 1000
_G = N // _BLK


def _dis_of(dega, degb):
    deg = dega + degb + 1.0
    return jnp.where(deg > 0, lax.rsqrt(jnp.maximum(deg, 1e-12)), 0.0)


def _y_body(x_ref, w_ref, dega_ref, degb_ref, y_ref):
    dis = _dis_of(dega_ref[...], degb_ref[...])
    y_ref[...] = (x_ref[...] @ w_ref[...]) * dis


def _mid_body(s1a_ref, s1b_ref, y1_ref, dega_ref, degb_ref, w2_ref, b1_ref,
              y2_ref):
    dis = _dis_of(dega_ref[...], degb_ref[...])
    h1 = dis * (s1a_ref[...] + s1b_ref[...] + y1_ref[...]) + b1_ref[...]
    h1 = jnp.maximum(h1, 0.0)
    y2_ref[...] = (h1 @ w2_ref[...]) * dis


def _pool_body(s2a_ref, s2b_ref, y2_ref, dega_ref, degb_ref, b2_ref,
               batch_ref, wlin_ref, blin_ref, out_ref, pooled_acc, counts_acc):
    i = pl.program_id(0)

    @pl.when(i == 0)
    def _():
        pooled_acc[...] = jnp.zeros_like(pooled_acc)
        counts_acc[...] = jnp.zeros_like(counts_acc)

    dis = _dis_of(dega_ref[...], degb_ref[...])
    h2 = dis * (s2a_ref[...] + s2b_ref[...] + y2_ref[...]) + b2_ref[...]
    gids = lax.broadcasted_iota(jnp.int32, (_BLK, NG), 1)
    oh = (batch_ref[...] == gids).astype(jnp.float32)
    dn = (((0,), (0,)), ((), ()))
    pooled_acc[...] += lax.dot_general(
        oh, h2, dn, preferred_element_type=jnp.float32)
    counts_acc[...] += lax.dot_general(
        oh, jnp.ones((_BLK, F), jnp.float32), dn,
        preferred_element_type=jnp.float32)

    @pl.when(i == _G - 1)
    def _():
        pooled = pooled_acc[...] / jnp.maximum(counts_acc[...], 1.0)
        out_ref[...] = pooled @ wlin_ref[...] + blin_ref[...]


def _row_spec(width):
    return pl.BlockSpec((_BLK, width), lambda i: (i, 0))


def _full_spec(shape):
    nd = len(shape)
    return pl.BlockSpec(shape, lambda i: (0,) * nd)


def kernel(x, edge_index, edge_weight, batch, W1, b1, W2, b2, Wlin, blin):
    pad_idx = jnp.arange(E_PAD - E, dtype=jnp.int32) % N
    row = jnp.concatenate([edge_index[0], pad_idx])
    col = jnp.concatenate([edge_index[1], pad_idx])
    ew = jnp.pad(edge_weight, (0, E_PAD - E))
    row2d = row.reshape(E_PAD // 128, 128)
    col2d = col.reshape(E_PAD // 128, 128)
    zeros1 = jnp.zeros((N,), jnp.float32)

    degp = _deg_kernel(col2d, ew, zeros1)
    dega = degp[0][:, None]
    degb = degp[1][:, None]

    y1 = pl.pallas_call(
        _y_body,
        grid=(_G,),
        in_specs=[_row_spec(F), _full_spec((F, F)), _row_spec(1), _row_spec(1)],
        out_specs=_row_spec(F),
        out_shape=jax.ShapeDtypeStruct((N, F), jnp.float32),
    )(x, W1, dega, degb)

    s1 = _edge_kernel(y1, row2d, col2d, ew)

    y2 = pl.pallas_call(
        _mid_body,
        grid=(_G,),
        in_specs=[_row_spec(F), _row_spec(F), _row_spec(F), _row_spec(1),
                  _row_spec(1), _full_spec((F, F)), _full_spec((1, F))],
        out_specs=_row_spec(F),
        out_shape=jax.ShapeDtypeStruct((N, F), jnp.float32),
    )(s1[0], s1[1], y1, dega, degb, W2, b1[None, :])

    s2 = _edge_kernel(y2, row2d, col2d, ew)

    out = pl.pallas_call(
        _pool_body,
        grid=(_G,),
        in_specs=[_row_spec(F), _row_spec(F), _row_spec(F), _row_spec(1),
                  _row_spec(1), _full_spec((1, F)), _row_spec(1),
                  _full_spec((F, NCLS)), _full_spec((1, NCLS))],
        out_specs=_full_spec((NG, NCLS)),
        out_shape=jax.ShapeDtypeStruct((NG, NCLS), jnp.float32),
        scratch_shapes=[pltpu.VMEM((NG, F), jnp.float32),
                        pltpu.VMEM((NG, F), jnp.float32)],
    )(s2[0], s2[1], y2, dega, degb, b2[None, :], batch[:, None],
      Wlin, blin[None, :])

    return out
